# baseline (speedup 1.0000x reference)
import functools

import jax
import jax.numpy as jnp
from jax import lax
from jax.experimental import pallas as pl
from jax.experimental.pallas import tpu as pltpu
from jax.experimental.pallas import tpu_sc as plsc

_B, _P = 131072, 256
_L = 8
_NC, _NS, _LANES = 2, 16, 16
_NW = _NC * _NS
_ROWS_PER_W = _B // _NW
_CH = 128
_NB = _CH // 8
_NCHUNK = _ROWS_PER_W // _CH
_NRING = 4

_mesh = plsc.VectorSubcoreMesh(core_axis_name="c", subcore_axis_name="s")


def _compute_chunk(in_v, w_vec, out_v):
    iota = lax.iota(jnp.int32, _LANES)
    for g in range(_CH // _LANES):
        rows = iota + (g * _LANES)
        band = rows // 8
        sub = rows % 8
        cols = [jnp.full((_LANES,), j, jnp.int32) for j in range(_L)]
        vs = [plsc.load_gather(in_v, [band, sub, cols[j]]) for j in range(_L)]
        sgn = [(-1.0 if j % 2 == 0 else 1.0) for j in range(_L)]
        cm = [vs[j] * sgn[j] for j in range(_L)]
        m = cm[0]
        for j in range(1, _L):
            m = jnp.maximum(m, cm[j])
        es = [jnp.exp(cm[j] - m) for j in range(_L)]
        s = es[0]
        for j in range(1, _L):
            s = s + es[j]
        scale = w_vec / s
        for j in range(_L):
            plsc.store_scatter(out_v, [rows, cols[j]], es[j] * (scale * sgn[j]))


@functools.partial(
    pl.kernel,
    mesh=_mesh,
    compiler_params=pltpu.CompilerParams(needs_layout_passes=False),
    out_type=jax.ShapeDtypeStruct((_B, _P), jnp.float32),
    scratch_types=(
        [pltpu.VMEM((_NB, 8, 128), jnp.float32) for _ in range(2)]
        + [pltpu.VMEM((_CH, 128), jnp.float32) for _ in range(_NRING)]
        + [
            pltpu.VMEM((_CH, 128), jnp.float32),
            pltpu.VMEM((_LANES,), jnp.float32),
        ]
        + [pltpu.SemaphoreType.DMA for _ in range(_NRING + 3)]
    ),
)
def _sc_kernel(in3_hbm, w_hbm, out_hbm,
               in_v0, in_v1, d0, d1, d2, d3, zt, w_v,
               os0, os1, os2, os3, zsem, is0, is1):
    wid = lax.axis_index("s") * _NC + lax.axis_index("c")
    row0 = wid * _ROWS_PER_W

    pltpu.sync_copy(w_hbm, w_v)
    w_vec = w_v[...]

    in_bufs = (in_v0, in_v1)
    data_tiles = (d0, d1, d2, d3)
    osems = (os0, os1, os2, os3)
    isems = (is0, is1)

    zero = jnp.zeros((_LANES,), jnp.float32)

    def _zero_row(r, _):
        for c in range(128 // _LANES):
            for t in data_tiles:
                t[r, pl.ds(c * _LANES, _LANES)] = zero
            zt[r, pl.ds(c * _LANES, _LANES)] = zero
        return _

    lax.fori_loop(0, _CH, _zero_row, None)

    def _fetch(chunk, b):
        band0 = (row0 + chunk * _CH) // 8
        pltpu.async_copy(
            in3_hbm.at[pl.ds(band0, _NB), :, pl.ds(0, 128)],
            in_bufs[b], isems[b])

    def _fetch_wait(chunk, b):
        band0 = (row0 + chunk * _CH) // 8
        pltpu.make_async_copy(
            in3_hbm.at[pl.ds(band0, _NB), :, pl.ds(0, 128)],
            in_bufs[b], isems[b]).wait()

    _fetch(0, 0)

    def _super(i, _):
        for u in range(_NRING):
            chunk = i * _NRING + u
            base = row0 + chunk * _CH
            ib = u % 2

            @pl.when(chunk + 1 < _NCHUNK)
            def _():
                _fetch(chunk + 1, 1 - ib)

            _fetch_wait(chunk, ib)

            @pl.when(i > 0)
            def _():
                pltpu.make_async_copy(
                    data_tiles[u], out_hbm.at[pl.ds(0, _CH), pl.ds(0, 128)],
                    osems[u]).wait()

            _compute_chunk(in_bufs[ib], w_vec, data_tiles[u])
            pltpu.async_copy(
                data_tiles[u],
                out_hbm.at[pl.ds(base, _CH), pl.ds(0, 128)], osems[u])
            pltpu.async_copy(
                zt, out_hbm.at[pl.ds(base, _CH), pl.ds(128, 128)], zsem)

            @pl.when(chunk >= 2)
            def _():
                pltpu.make_async_copy(
                    zt, out_hbm.at[pl.ds(0, _CH), pl.ds(128, 128)], zsem
                ).wait()
        return _

    lax.fori_loop(0, _NCHUNK // _NRING, _super, None)
    for u in range(_NRING):
        pltpu.make_async_copy(
            data_tiles[u], out_hbm.at[pl.ds(0, _CH), pl.ds(0, 128)], osems[u]
        ).wait()
    for _ in range(2):
        pltpu.make_async_copy(
            zt, out_hbm.at[pl.ds(0, _CH), pl.ds(128, 128)], zsem).wait()


@jax.jit
def kernel(inputs, clause_weight):
    in3 = inputs.reshape(_B // 8, 8, _P)
    w16 = jnp.broadcast_to(clause_weight.reshape(()), (_LANES,))
    return _sc_kernel(in3, w16)
